# Initial kernel scaffold; baseline (speedup 1.0000x reference)
import jax
import jax.numpy as jnp
from jax.experimental import pallas as pl


def kernel(x, W_enc, b_enc, W_dec, b_dec):
    raise NotImplementedError("write your pallas kernel here")


# speedup vs baseline: 2.6756x; 2.6756x over previous
import jax
import jax.numpy as jnp
from jax import lax
from jax.experimental import pallas as pl
from jax.experimental.pallas import tpu as pltpu

DIN = 768
NFEAT = 16384
KTOP = 32
NTOK = 2048

RBLK = 128
FT_ENC = 1024
FT_DEC = 512
NSTRIP = NFEAT // FT_ENC
NEG = -jnp.inf


def _encode_topk_kernel(x_ref, we_ref, be_ref, h_ref):
    f = pl.program_id(1)
    nf = pl.num_programs(1)
    z_part = (
        jnp.dot(x_ref[...], we_ref[...], preferred_element_type=jnp.float32)
        + be_ref[...]
    )
    h_ref[:, pl.ds(f * FT_ENC, FT_ENC)] = z_part

    @pl.when(f == nf - 1)
    def _finalize():
        def iter_body(i, m):
            def strip_body(s, acc):
                zs = h_ref[:, pl.ds(s * FT_ENC, FT_ENC)]
                ms = jnp.max(
                    jnp.where(zs < m, zs, NEG), axis=1, keepdims=True
                )
                return jnp.maximum(acc, ms)

            return lax.fori_loop(0, NSTRIP, strip_body,
                                 jnp.full((RBLK, 1), NEG, jnp.float32))

        t = lax.fori_loop(0, KTOP, iter_body,
                          jnp.full((RBLK, 1), jnp.inf, jnp.float32))

        def mask_body(s, carry):
            zs = h_ref[:, pl.ds(s * FT_ENC, FT_ENC)]
            h_ref[:, pl.ds(s * FT_ENC, FT_ENC)] = jnp.where(
                zs >= t, jnp.maximum(zs, 0.0), 0.0
            )
            return carry

        lax.fori_loop(0, NSTRIP, mask_body, 0)


def _decode_kernel(h_ref, wd_ref, bd_ref, out_ref):
    f = pl.program_id(1)

    @pl.when(f == 0)
    def _init():
        out_ref[...] = jnp.broadcast_to(bd_ref[...], (RBLK, DIN))

    out_ref[...] += jnp.dot(h_ref[...], wd_ref[...],
                            preferred_element_type=jnp.float32)


def kernel(x, W_enc, b_enc, W_dec, b_dec):
    h = pl.pallas_call(
        _encode_topk_kernel,
        grid=(NTOK // RBLK, NFEAT // FT_ENC),
        in_specs=[
            pl.BlockSpec((RBLK, DIN), lambda t, f: (t, 0)),
            pl.BlockSpec((DIN, FT_ENC), lambda t, f: (0, f)),
            pl.BlockSpec((1, FT_ENC), lambda t, f: (0, f)),
        ],
        out_specs=pl.BlockSpec((RBLK, NFEAT), lambda t, f: (t, 0)),
        out_shape=jax.ShapeDtypeStruct((NTOK, NFEAT), jnp.float32),
        compiler_params=pltpu.CompilerParams(
            dimension_semantics=("parallel", "arbitrary"),
        ),
    )(x, W_enc, b_enc.reshape(1, NFEAT))

    recon = pl.pallas_call(
        _decode_kernel,
        grid=(NTOK // RBLK, NFEAT // FT_DEC),
        in_specs=[
            pl.BlockSpec((RBLK, FT_DEC), lambda t, f: (t, f)),
            pl.BlockSpec((FT_DEC, DIN), lambda t, f: (f, 0)),
            pl.BlockSpec((1, DIN), lambda t, f: (0, 0)),
        ],
        out_specs=pl.BlockSpec((RBLK, DIN), lambda t, f: (t, 0)),
        out_shape=jax.ShapeDtypeStruct((NTOK, DIN), jnp.float32),
        compiler_params=pltpu.CompilerParams(
            dimension_semantics=("parallel", "arbitrary"),
        ),
    )(h, W_dec, b_dec.reshape(1, DIN))
    return recon
